# Initial kernel scaffold; baseline (speedup 1.0000x reference)
import functools

import jax
import jax.numpy as jnp
from jax import lax
from jax.experimental import pallas as pl
from jax.experimental.pallas import tpu as pltpu
from jax.experimental.pallas import tpu_sc as plsc

N_NODES = 10000
N_EDGES = 320000
D = 128

NC = 2
NS = 16
NW = NC * NS
LANES = 16

B = 128
K = 79
E_PAD = NW * K * B
N_PAD = 10240
ROWS_PER_TILE = N_PAD // NS


def _sc_segment_sum(x, src2, dst2):
    mesh = plsc.VectorSubcoreMesh(core_axis_name="c", subcore_axis_name="s")

    @functools.partial(
        pl.kernel,
        out_type=(
            jax.ShapeDtypeStruct((NC, N_PAD, D), jnp.float32),
            jax.ShapeDtypeStruct((NC, N_PAD, LANES), jnp.float32),
        ),
        mesh=mesh,
        scratch_types=[
            pltpu.VMEM((K, B), jnp.int32),
            pltpu.VMEM((K, B), jnp.int32),
            pltpu.VMEM((B, D), jnp.float32),
            pltpu.VMEM((B, LANES), jnp.float32),
            pltpu.VMEM((B, LANES), jnp.float32),
            pltpu.VMEM_SHARED((N_PAD, D), jnp.float32),
            pltpu.VMEM_SHARED((N_PAD, LANES), jnp.float32),
            pltpu.SemaphoreType.DMA,
        ],
    )
    def sc_kernel(x_hbm, src_hbm, dst_hbm, sums_out, cnt_out,
                  src_v, dst_v, rows_v, ones_v, zc_v, sums_sh, cnt_sh, sem):
        cid = lax.axis_index("c")
        sid = lax.axis_index("s")
        wid = sid * NC + cid

        pltpu.sync_copy(src_hbm.at[wid], src_v)
        pltpu.sync_copy(dst_hbm.at[wid], dst_v)

        @pl.loop(0, B)
        def _(i):
            ones_v[i, :] = jnp.ones((LANES,), jnp.float32)
            zc_v[i, :] = jnp.zeros((LANES,), jnp.float32)

        @pl.loop(0, B)
        def _(i):
            @pl.loop(0, D, step=LANES)
            def _(j):
                rows_v[i, pl.ds(j, LANES)] = jnp.zeros((LANES,), jnp.float32)

        base = sid * ROWS_PER_TILE
        for r in range(ROWS_PER_TILE // B):
            pltpu.sync_copy(rows_v, sums_sh.at[pl.ds(base + r * B, B)])
            pltpu.sync_copy(zc_v, cnt_sh.at[pl.ds(base + r * B, B)])
        plsc.subcore_barrier()

        @pl.loop(0, K)
        def _(j):
            pltpu.async_copy(x_hbm.at[src_v.at[j]], rows_v, sem).wait()
            pltpu.sync_copy(rows_v, sums_sh.at[dst_v.at[j]], add=True)
            pltpu.sync_copy(ones_v, cnt_sh.at[dst_v.at[j]], add=True)

        plsc.subcore_barrier()

        pltpu.sync_copy(sums_sh.at[pl.ds(base, ROWS_PER_TILE)],
                        sums_out.at[cid, pl.ds(base, ROWS_PER_TILE)])
        pltpu.sync_copy(cnt_sh.at[pl.ds(base, ROWS_PER_TILE)],
                        cnt_out.at[cid, pl.ds(base, ROWS_PER_TILE)])

    return sc_kernel(x, src2, dst2)


_BLK = 1000


def _tc_body(s_ref, c_ref, x_ref, wl_ref, wr_ref, b_ref, o_ref):
    cnt = c_ref[0, :, :1] + c_ref[1, :, :1]
    agg = (s_ref[0] + s_ref[1]) / jnp.maximum(cnt, 1.0)
    o_ref[...] = (
        jnp.dot(agg, wl_ref[...], preferred_element_type=jnp.float32)
        + jnp.dot(x_ref[...], wr_ref[...], preferred_element_type=jnp.float32)
        + b_ref[...]
    )


def _tc_combine(sums, cnts, x, W_l, W_r, b):
    grid = (N_NODES // _BLK,)
    return pl.pallas_call(
        _tc_body,
        grid=grid,
        in_specs=[
            pl.BlockSpec((NC, _BLK, D), lambda i: (0, i, 0)),
            pl.BlockSpec((NC, _BLK, LANES), lambda i: (0, i, 0)),
            pl.BlockSpec((_BLK, D), lambda i: (i, 0)),
            pl.BlockSpec((D, D), lambda i: (0, 0)),
            pl.BlockSpec((D, D), lambda i: (0, 0)),
            pl.BlockSpec((1, D), lambda i: (0, 0)),
        ],
        out_specs=pl.BlockSpec((_BLK, D), lambda i: (i, 0)),
        out_shape=jax.ShapeDtypeStruct((N_NODES, D), jnp.float32),
    )(sums, cnts, x, W_l, W_r, b)


def kernel(x, edge_index, W_l, b_l, W_r, b_r):
    ei = edge_index.astype(jnp.int32)
    pad = E_PAD - N_EDGES
    src = jnp.concatenate([ei[0], jnp.zeros((pad,), jnp.int32)])
    dst = jnp.concatenate([ei[1], jnp.full((pad,), N_NODES, jnp.int32)])
    src2 = src.reshape(NW, K, B)
    dst2 = dst.reshape(NW, K, B)

    sums, cnts = _sc_segment_sum(x, src2, dst2)
    b = (b_l + b_r).reshape(1, D)
    return _tc_combine(sums, cnts, x, W_l, W_r, b)


# speedup vs baseline: 3.9478x; 3.9478x over previous
import functools

import jax
import jax.numpy as jnp
from jax import lax
from jax.experimental import pallas as pl
from jax.experimental.pallas import tpu as pltpu
from jax.experimental.pallas import tpu_sc as plsc

N_NODES = 10000
N_EDGES = 320000
D = 128

NC = 2
NS = 16
NW = NC * NS
LANES = 16

B = 128
K = 80
ICH = 16
E_PAD = NW * K * B
N_PAD = 10240
ROWS_PER_TILE = N_PAD // NS


def _sc_segment_sum(x, src2, dst2):
    mesh = plsc.VectorSubcoreMesh(core_axis_name="c", subcore_axis_name="s")

    @functools.partial(
        pl.kernel,
        out_type=(
            jax.ShapeDtypeStruct((NC, N_PAD, D), jnp.float32),
            jax.ShapeDtypeStruct((NC, N_PAD, LANES), jnp.float32),
        ),
        mesh=mesh,
        scratch_types=[
            pltpu.VMEM((ICH, B), jnp.int32),
            pltpu.VMEM((ICH, B), jnp.int32),
            pltpu.VMEM((B, D), jnp.float32),
            pltpu.VMEM((B, LANES), jnp.float32),
            pltpu.VMEM((B, LANES), jnp.float32),
            pltpu.VMEM_SHARED((N_PAD, D), jnp.float32),
            pltpu.VMEM_SHARED((N_PAD, LANES), jnp.float32),
            pltpu.SemaphoreType.DMA,
        ],
        compiler_params=pltpu.CompilerParams(use_tc_tiling_on_sc=False),
    )
    def sc_kernel(x_hbm, src_hbm, dst_hbm, sums_out, cnt_out,
                  src_v, dst_v, rows_v, ones_v, zc_v, sums_sh, cnt_sh, sem):
        cid = lax.axis_index("c")
        sid = lax.axis_index("s")
        wid = sid * NC + cid

        @pl.loop(0, B)
        def _(i):
            ones_v[i, :] = jnp.ones((LANES,), jnp.float32)
            zc_v[i, :] = jnp.zeros((LANES,), jnp.float32)

        @pl.loop(0, B)
        def _(i):
            @pl.loop(0, D, step=LANES)
            def _(j):
                rows_v[i, pl.ds(j, LANES)] = jnp.zeros((LANES,), jnp.float32)

        base = sid * ROWS_PER_TILE
        for r in range(ROWS_PER_TILE // B):
            pltpu.sync_copy(rows_v, sums_sh.at[pl.ds(base + r * B, B)])
            pltpu.sync_copy(zc_v, cnt_sh.at[pl.ds(base + r * B, B)])
        plsc.subcore_barrier()

        @pl.loop(0, K // ICH)
        def _(g):
            pltpu.sync_copy(src_hbm.at[wid, pl.ds(g * ICH, ICH)], src_v)
            pltpu.sync_copy(dst_hbm.at[wid, pl.ds(g * ICH, ICH)], dst_v)

            @pl.loop(0, ICH)
            def _(jj):
                pltpu.async_copy(x_hbm.at[src_v.at[jj]], rows_v, sem).wait()
                pltpu.sync_copy(rows_v, sums_sh.at[dst_v.at[jj]], add=True)
                pltpu.sync_copy(ones_v, cnt_sh.at[dst_v.at[jj]], add=True)

        plsc.subcore_barrier()

        pltpu.sync_copy(sums_sh.at[pl.ds(base, ROWS_PER_TILE)],
                        sums_out.at[cid, pl.ds(base, ROWS_PER_TILE)])
        pltpu.sync_copy(cnt_sh.at[pl.ds(base, ROWS_PER_TILE)],
                        cnt_out.at[cid, pl.ds(base, ROWS_PER_TILE)])

    return sc_kernel(x, src2, dst2)


_BLK = 1000


def _tc_body(s_ref, c_ref, x_ref, wl_ref, wr_ref, b_ref, o_ref):
    cnt = c_ref[0, :, :1] + c_ref[1, :, :1]
    agg = (s_ref[0] + s_ref[1]) / jnp.maximum(cnt, 1.0)
    o_ref[...] = (
        jnp.dot(agg, wl_ref[...], preferred_element_type=jnp.float32)
        + jnp.dot(x_ref[...], wr_ref[...], preferred_element_type=jnp.float32)
        + b_ref[...]
    )


def _tc_combine(sums, cnts, x, W_l, W_r, b):
    grid = (N_NODES // _BLK,)
    return pl.pallas_call(
        _tc_body,
        grid=grid,
        in_specs=[
            pl.BlockSpec((NC, _BLK, D), lambda i: (0, i, 0)),
            pl.BlockSpec((NC, _BLK, LANES), lambda i: (0, i, 0)),
            pl.BlockSpec((_BLK, D), lambda i: (i, 0)),
            pl.BlockSpec((D, D), lambda i: (0, 0)),
            pl.BlockSpec((D, D), lambda i: (0, 0)),
            pl.BlockSpec((1, D), lambda i: (0, 0)),
        ],
        out_specs=pl.BlockSpec((_BLK, D), lambda i: (i, 0)),
        out_shape=jax.ShapeDtypeStruct((N_NODES, D), jnp.float32),
    )(sums, cnts, x, W_l, W_r, b)


def kernel(x, edge_index, W_l, b_l, W_r, b_r):
    ei = edge_index.astype(jnp.int32)
    pad = E_PAD - N_EDGES
    src = jnp.concatenate([ei[0], jnp.zeros((pad,), jnp.int32)])
    dst = jnp.concatenate([ei[1], jnp.full((pad,), N_NODES, jnp.int32)])
    src2 = src.reshape(NW, K, B)
    dst2 = dst.reshape(NW, K, B)

    sums, cnts = _sc_segment_sum(x, src2, dst2)
    b = (b_l + b_r).reshape(1, D)
    return _tc_combine(sums, cnts, x, W_l, W_r, b)


# speedup vs baseline: 4.6926x; 1.1887x over previous
import functools

import jax
import jax.numpy as jnp
from jax import lax
from jax.experimental import pallas as pl
from jax.experimental.pallas import tpu as pltpu
from jax.experimental.pallas import tpu_sc as plsc

N_NODES = 10000
N_EDGES = 320000
D = 128
DA = D + 16

NC = 2
NS = 16
NW = NC * NS
LANES = 16

B = 128
K = 80
HC = 5
NHC = K // HC
E_PAD = NW * K * B
N_PAD = 10016
ROWS_PER_TILE = N_PAD // NS


def _sc_segment_sum(xa, src3, dst3):
    mesh = plsc.VectorSubcoreMesh(core_axis_name="c", subcore_axis_name="s")

    @functools.partial(
        pl.kernel,
        out_type=jax.ShapeDtypeStruct((NC, N_PAD, DA), jnp.float32),
        mesh=mesh,
        scratch_types=[
            pltpu.VMEM((2 * HC, B), jnp.int32),
            pltpu.VMEM((2 * HC, B), jnp.int32),
            pltpu.VMEM((B, DA), jnp.float32),
            pltpu.VMEM((B, DA), jnp.float32),
            pltpu.VMEM_SHARED((N_PAD, DA), jnp.float32),
            pltpu.SemaphoreType.DMA,
            pltpu.SemaphoreType.DMA,
            pltpu.SemaphoreType.DMA,
            pltpu.SemaphoreType.DMA,
        ],
        compiler_params=pltpu.CompilerParams(use_tc_tiling_on_sc=False),
    )
    def sc_kernel(xa_hbm, src_hbm, dst_hbm, sums_out,
                  src_v, dst_v, rows0, rows1, sums_sh, g0, g1, s0, s1):
        cid = lax.axis_index("c")
        sid = lax.axis_index("s")
        wid = sid * NC + cid
        rows = (rows0, rows1)
        gsem = (g0, g1)
        ssem = (s0, s1)

        def stage(hc):
            off = lax.rem(hc, 2) * HC
            pltpu.sync_copy(src_hbm.at[wid, hc], src_v.at[pl.ds(off, HC)])
            pltpu.sync_copy(dst_hbm.at[wid, hc], dst_v.at[pl.ds(off, HC)])

        def issue_gather(j, p):
            r = lax.rem(j, 2 * HC)
            pltpu.async_copy(xa_hbm.at[src_v.at[r]], rows[p], gsem[p])

        def wait_gather(j, p):
            r = lax.rem(j, 2 * HC)
            pltpu.make_async_copy(xa_hbm.at[src_v.at[r]], rows[p],
                                  gsem[p]).wait()

        def issue_scatter(j, p):
            r = lax.rem(j, 2 * HC)
            pltpu.async_copy(rows[p], sums_sh.at[dst_v.at[r]], ssem[p],
                             add=True)

        def wait_scatter(j, p):
            r = lax.rem(j, 2 * HC)
            pltpu.make_async_copy(rows[p], sums_sh.at[dst_v.at[r]],
                                  ssem[p]).wait()

        @pl.loop(0, B)
        def _(i):
            @pl.loop(0, DA, step=LANES)
            def _(j):
                rows0[i, pl.ds(j, LANES)] = jnp.zeros((LANES,), jnp.float32)

        base = sid * ROWS_PER_TILE
        for r in range(ROWS_PER_TILE // B):
            pltpu.sync_copy(rows0, sums_sh.at[pl.ds(base + r * B, B)])
        tail = ROWS_PER_TILE % B
        pltpu.sync_copy(
            rows0.at[pl.ds(0, tail)],
            sums_sh.at[pl.ds(base + (ROWS_PER_TILE // B) * B, tail)])
        plsc.subcore_barrier()

        stage(0)
        issue_gather(0, 0)

        @pl.loop(0, K, step=2)
        def _(j):
            wait_gather(j, 0)
            issue_scatter(j, 0)

            @pl.when(j > 0)
            def _():
                wait_scatter(j - 1, 1)

            @pl.when(lax.rem(j + 1, HC) == 0)
            def _():
                stage((j + 1) // HC)

            issue_gather(j + 1, 1)

            wait_gather(j + 1, 1)
            issue_scatter(j + 1, 1)
            wait_scatter(j, 0)

            @pl.when(j + 2 < K)
            def _():
                @pl.when(lax.rem(j + 2, HC) == 0)
                def _():
                    stage((j + 2) // HC)

                issue_gather(j + 2, 0)

        wait_scatter(K - 1, 1)
        plsc.subcore_barrier()

        pltpu.sync_copy(sums_sh.at[pl.ds(base, ROWS_PER_TILE)],
                        sums_out.at[cid, pl.ds(base, ROWS_PER_TILE)])

    return sc_kernel(xa, src3, dst3)


_BLK = 1000


def _tc_body(s_ref, x_ref, wl_ref, wr_ref, b_ref, o_ref):
    s = s_ref[0] + s_ref[1]
    agg = s[:, :D] / jnp.maximum(s[:, D:D + 1], 1.0)
    o_ref[...] = (
        jnp.dot(agg, wl_ref[...], preferred_element_type=jnp.float32)
        + jnp.dot(x_ref[...], wr_ref[...], preferred_element_type=jnp.float32)
        + b_ref[...]
    )


def _tc_combine(sums, x, W_l, W_r, b):
    grid = (N_NODES // _BLK,)
    return pl.pallas_call(
        _tc_body,
        grid=grid,
        in_specs=[
            pl.BlockSpec((NC, _BLK, DA), lambda i: (0, i, 0)),
            pl.BlockSpec((_BLK, D), lambda i: (i, 0)),
            pl.BlockSpec((D, D), lambda i: (0, 0)),
            pl.BlockSpec((D, D), lambda i: (0, 0)),
            pl.BlockSpec((1, D), lambda i: (0, 0)),
        ],
        out_specs=pl.BlockSpec((_BLK, D), lambda i: (i, 0)),
        out_shape=jax.ShapeDtypeStruct((N_NODES, D), jnp.float32),
    )(sums, x, W_l, W_r, b)


def kernel(x, edge_index, W_l, b_l, W_r, b_r):
    ei = edge_index.astype(jnp.int32)
    pad = E_PAD - N_EDGES
    src = jnp.concatenate([ei[0], jnp.zeros((pad,), jnp.int32)])
    dst = jnp.concatenate([ei[1], jnp.full((pad,), N_NODES, jnp.int32)])
    src3 = src.reshape(NW, NHC, HC, B)
    dst3 = dst.reshape(NW, NHC, HC, B)
    xa = jnp.concatenate([x, jnp.ones((N_NODES, LANES), jnp.float32)], axis=1)

    sums = _sc_segment_sum(xa, src3, dst3)
    b = (b_l + b_r).reshape(1, D)
    return _tc_combine(sums, x, W_l, W_r, b)
